# baseline (speedup 1.0000x reference)
import functools

import jax
import jax.numpy as jnp
from jax import lax
from jax.experimental import pallas as pl
from jax.experimental.pallas import tpu as pltpu
from jax.experimental.pallas import tpu_sc as plsc

_NC = 2
_NS = 16
_NW = _NC * _NS

_EMBED = 64
_OUT = 512

_IDXW = 128
_SUB = 8
_CHUNK = _IDXW * _SUB


def _gather_kernel(n_tokens: int, idx, table):
    b_per_w = n_tokens // _NW
    n_chunks = b_per_w // _CHUNK
    mesh = plsc.VectorSubcoreMesh(core_axis_name="c", subcore_axis_name="s")

    @functools.partial(
        pl.kernel,
        out_type=jax.ShapeDtypeStruct((n_tokens, _EMBED), jnp.float32),
        mesh=mesh,
        scratch_types=[
            pltpu.VMEM((_SUB, _IDXW), jnp.int32),
            pltpu.VMEM((_CHUNK, _EMBED), jnp.float32),
            pltpu.SemaphoreType.DMA,
        ],
        compiler_params=pltpu.CompilerParams(use_tc_tiling_on_sc=False),
    )
    def gk(idx_hbm, table_hbm, out_hbm, idx_v, rows_v, sem):
        wid = lax.axis_index("s") * _NC + lax.axis_index("c")
        w_base = wid * b_per_w

        def body(i, carry):
            base = w_base + i * _CHUNK
            pltpu.sync_copy(
                idx_hbm.at[pl.ds(pl.multiple_of(base // _IDXW, _SUB), _SUB)],
                idx_v.at[...],
            )
            for j in range(_SUB):
                pltpu.async_copy(
                    table_hbm.at[idx_v.at[j]],
                    rows_v.at[pl.ds(j * _IDXW, _IDXW)],
                    sem,
                )
            for j in range(_SUB):
                pltpu.make_async_copy(
                    table_hbm.at[idx_v.at[j]],
                    rows_v.at[pl.ds(j * _IDXW, _IDXW)],
                    sem,
                ).wait()
            pltpu.sync_copy(rows_v.at[...], out_hbm.at[pl.ds(base, _CHUNK)])
            return carry

        lax.fori_loop(0, n_chunks, body, 0, unroll=False)

    return gk(idx.reshape(n_tokens // _IDXW, _IDXW), table)


def _matmul_kernel(emb, W, b2d, n_tokens: int, bm: int):
    def mk(e_ref, w_ref, b_ref, o_ref):
        acc = jnp.dot(e_ref[...], w_ref[...], preferred_element_type=jnp.float32)
        o_ref[...] = jnp.maximum(acc + b_ref[...], 0.0)

    return pl.pallas_call(
        mk,
        grid=(n_tokens // bm,),
        in_specs=[
            pl.BlockSpec((bm, _EMBED), lambda i: (i, 0)),
            pl.BlockSpec((_EMBED, _OUT), lambda i: (0, 0)),
            pl.BlockSpec((1, _OUT), lambda i: (0, 0)),
        ],
        out_specs=pl.BlockSpec((bm, _OUT), lambda i: (i, 0)),
        out_shape=jax.ShapeDtypeStruct((n_tokens, _OUT), jnp.float32),
    )(emb, W, b2d)


def kernel(x, table, W, b):
    batch, hist = x.shape
    n_tokens = batch * hist
    idx = x.reshape(n_tokens)
    emb = _gather_kernel(n_tokens, idx, table)
    out = _matmul_kernel(emb, W, b.reshape(1, _OUT), n_tokens, 2048)
    return out.reshape(batch, hist, _OUT)


# speedup vs baseline: 1.0478x; 1.0478x over previous
import functools

import jax
import jax.numpy as jnp
from jax import lax
from jax.experimental import pallas as pl
from jax.experimental.pallas import tpu as pltpu
from jax.experimental.pallas import tpu_sc as plsc

_NC = 2
_NS = 16
_NW = _NC * _NS

_EMBED = 64
_LANE = 128
_OUT = 512

_IDXW = 128
_SUB = 4
_CHUNK = _IDXW * _SUB


def _gather_kernel(n_tokens: int, idx, table128):
    b_per_w = n_tokens // _NW
    n_chunks = b_per_w // _CHUNK
    mesh = plsc.VectorSubcoreMesh(core_axis_name="c", subcore_axis_name="s")

    @functools.partial(
        pl.kernel,
        out_type=jax.ShapeDtypeStruct((n_tokens, _LANE), jnp.float32),
        mesh=mesh,
        scratch_types=[
            pltpu.VMEM((_SUB, _IDXW), jnp.int32),
            pltpu.VMEM((_CHUNK, _LANE), jnp.float32),
            pltpu.SemaphoreType.DMA,
        ],
    )
    def gk(idx_hbm, table_hbm, out_hbm, idx_v, rows_v, sem):
        wid = lax.axis_index("s") * _NC + lax.axis_index("c")
        w_base = wid * b_per_w

        def body(i, carry):
            base = w_base + i * _CHUNK
            pltpu.sync_copy(
                idx_hbm.at[pl.ds(pl.multiple_of(base // _IDXW, _SUB), _SUB)],
                idx_v.at[...],
            )
            for j in range(_SUB):
                pltpu.async_copy(
                    table_hbm.at[idx_v.at[j]],
                    rows_v.at[pl.ds(j * _IDXW, _IDXW)],
                    sem,
                )
            for j in range(_SUB):
                pltpu.make_async_copy(
                    table_hbm.at[idx_v.at[j]],
                    rows_v.at[pl.ds(j * _IDXW, _IDXW)],
                    sem,
                ).wait()
            pltpu.sync_copy(rows_v.at[...], out_hbm.at[pl.ds(base, _CHUNK)])
            return carry

        lax.fori_loop(0, n_chunks, body, 0, unroll=False)

    return gk(idx.reshape(n_tokens // _IDXW, _IDXW), table128)


def _matmul_kernel(emb, W, b2d, n_tokens: int, bm: int):
    def mk(e_ref, w_ref, b_ref, o_ref):
        acc = jnp.dot(
            e_ref[:, :_EMBED], w_ref[...], preferred_element_type=jnp.float32
        )
        o_ref[...] = jnp.maximum(acc + b_ref[...], 0.0)

    return pl.pallas_call(
        mk,
        grid=(n_tokens // bm,),
        in_specs=[
            pl.BlockSpec((bm, _LANE), lambda i: (i, 0)),
            pl.BlockSpec((_EMBED, _OUT), lambda i: (0, 0)),
            pl.BlockSpec((1, _OUT), lambda i: (0, 0)),
        ],
        out_specs=pl.BlockSpec((bm, _OUT), lambda i: (i, 0)),
        out_shape=jax.ShapeDtypeStruct((n_tokens, _OUT), jnp.float32),
    )(emb, W, b2d)


def kernel(x, table, W, b):
    batch, hist = x.shape
    n_tokens = batch * hist
    idx = x.reshape(n_tokens)
    table128 = jnp.pad(table, ((0, 0), (0, _LANE - _EMBED)))
    emb = _gather_kernel(n_tokens, idx, table128)
    out = _matmul_kernel(emb, W, b.reshape(1, _OUT), n_tokens, 2048)
    return out.reshape(batch, hist, _OUT)


# speedup vs baseline: 3.0515x; 2.9123x over previous
import functools

import jax
import jax.numpy as jnp
from jax import lax
from jax.experimental import pallas as pl
from jax.experimental.pallas import tpu as pltpu
from jax.experimental.pallas import tpu_sc as plsc

_NC = 2
_NS = 16
_NW = _NC * _NS

_EMBED = 64
_LANE = 128
_OUT = 512

_IDXW = 128
_SUB = 4
_CHUNK = _IDXW * _SUB


def _gather_kernel(n_tokens: int, idx, table128):
    b_per_w = n_tokens // _NW
    n_chunks = b_per_w // _CHUNK
    mesh = plsc.VectorSubcoreMesh(core_axis_name="c", subcore_axis_name="s")

    @functools.partial(
        pl.kernel,
        out_type=jax.ShapeDtypeStruct((n_tokens, _LANE), jnp.float32),
        mesh=mesh,
        scratch_types=[
            pltpu.VMEM((_SUB, _IDXW), jnp.int32),
            pltpu.VMEM((_CHUNK, _LANE), jnp.float32),
            pltpu.SemaphoreType.DMA,
        ],
    )
    def gk(idx_hbm, table_hbm, out_hbm, idx_v, rows_v, sem):
        wid = lax.axis_index("s") * _NC + lax.axis_index("c")
        w_base = wid * b_per_w

        def body(i, carry):
            base = w_base + i * _CHUNK
            pltpu.sync_copy(
                idx_hbm.at[pl.ds(pl.multiple_of(base // _IDXW, _SUB), _SUB)],
                idx_v.at[...],
            )
            for j in range(_SUB):
                pltpu.async_copy(
                    table_hbm.at[idx_v.at[j]],
                    rows_v.at[pl.ds(j * _IDXW, _IDXW)],
                    sem,
                )
            for j in range(_SUB):
                pltpu.make_async_copy(
                    table_hbm.at[idx_v.at[j]],
                    rows_v.at[pl.ds(j * _IDXW, _IDXW)],
                    sem,
                ).wait()
            pltpu.sync_copy(rows_v.at[...], out_hbm.at[pl.ds(base, _CHUNK)])
            return carry

        lax.fori_loop(0, n_chunks, body, 0, unroll=False)

    return gk(idx.reshape(n_tokens // _IDXW, _IDXW), table128)


def _transpose_pad_kernel(tableT, n_rows: int, bn: int):
    def tk(t_ref, o_ref):
        t = jnp.transpose(t_ref[...], (1, 0))
        o_ref[...] = jnp.concatenate([t, t], axis=1)

    grid = (n_rows + bn - 1) // bn
    return pl.pallas_call(
        tk,
        grid=(grid,),
        in_specs=[pl.BlockSpec((_EMBED, bn), lambda i: (0, i))],
        out_specs=pl.BlockSpec((bn, _LANE), lambda i: (i, 0)),
        out_shape=jax.ShapeDtypeStruct((n_rows, _LANE), jnp.float32),
    )(tableT)


def _matmul_kernel(emb, W, b2d, n_tokens: int, bm: int):
    def mk(e_ref, w_ref, b_ref, o_ref):
        acc = jnp.dot(
            e_ref[:, :_EMBED], w_ref[...], preferred_element_type=jnp.float32
        )
        o_ref[...] = jnp.maximum(acc + b_ref[...], 0.0)

    return pl.pallas_call(
        mk,
        grid=(n_tokens // bm,),
        in_specs=[
            pl.BlockSpec((bm, _LANE), lambda i: (i, 0)),
            pl.BlockSpec((_EMBED, _OUT), lambda i: (0, 0)),
            pl.BlockSpec((1, _OUT), lambda i: (0, 0)),
        ],
        out_specs=pl.BlockSpec((bm, _OUT), lambda i: (i, 0)),
        out_shape=jax.ShapeDtypeStruct((n_tokens, _OUT), jnp.float32),
    )(emb, W, b2d)


def kernel(x, table, W, b):
    batch, hist = x.shape
    n_tokens = batch * hist
    n_rows = table.shape[0]
    idx = jnp.transpose(x).reshape(n_tokens)
    table128 = _transpose_pad_kernel(jnp.transpose(table), n_rows, 4096)
    emb = _gather_kernel(n_tokens, idx, table128)
    out = _matmul_kernel(emb, W, b.reshape(1, _OUT), n_tokens, 2048)
    return jnp.transpose(out.reshape(hist, batch, _OUT), (1, 0, 2))


# speedup vs baseline: 3.3663x; 1.1032x over previous
import functools

import jax
import jax.numpy as jnp
from jax import lax
from jax.experimental import pallas as pl
from jax.experimental.pallas import tpu as pltpu
from jax.experimental.pallas import tpu_sc as plsc

_NC = 2
_NS = 16
_NW = _NC * _NS

_EMBED = 64
_LANE = 128
_OUT = 512

_IDXW = 128
_SUB = 4
_CHUNK = _IDXW * _SUB


def _gather_kernel(n_tokens: int, idx, table128):
    b_per_w = n_tokens // _NW
    n_chunks = b_per_w // _CHUNK
    mesh = plsc.VectorSubcoreMesh(core_axis_name="c", subcore_axis_name="s")

    @functools.partial(
        pl.kernel,
        out_type=jax.ShapeDtypeStruct((n_tokens, _LANE), jnp.float32),
        mesh=mesh,
        scratch_types=[
            pltpu.VMEM((_SUB, _IDXW), jnp.int32),
            pltpu.VMEM((_CHUNK, _LANE), jnp.float32),
            pltpu.SemaphoreType.DMA,
        ],
    )
    def gk(idx_hbm, table_hbm, out_hbm, idx_v, rows_v, sem):
        wid = lax.axis_index("s") * _NC + lax.axis_index("c")
        w_base = wid * b_per_w

        def body(i, carry):
            base = w_base + i * _CHUNK
            pltpu.sync_copy(
                idx_hbm.at[pl.ds(pl.multiple_of(base // _IDXW, _SUB), _SUB)],
                idx_v.at[...],
            )
            for j in range(_SUB):
                pltpu.async_copy(
                    table_hbm.at[idx_v.at[j]],
                    rows_v.at[pl.ds(j * _IDXW, _IDXW)],
                    sem,
                )
            for j in range(_SUB):
                pltpu.make_async_copy(
                    table_hbm.at[idx_v.at[j]],
                    rows_v.at[pl.ds(j * _IDXW, _IDXW)],
                    sem,
                ).wait()
            pltpu.sync_copy(rows_v.at[...], out_hbm.at[pl.ds(base, _CHUNK)])
            return carry

        lax.fori_loop(0, n_chunks, body, 0, unroll=False)

    return gk(idx.reshape(n_tokens // _IDXW, _IDXW), table128)


def _transpose_pad_kernel(tableT, n_rows: int, bn: int):
    def tk(t_ref, o_ref):
        t = jnp.transpose(t_ref[...], (1, 0))
        o_ref[...] = jnp.concatenate([t, t], axis=1)

    grid = (n_rows + bn - 1) // bn
    return pl.pallas_call(
        tk,
        grid=(grid,),
        in_specs=[pl.BlockSpec((_EMBED, bn), lambda i: (0, i))],
        out_specs=pl.BlockSpec((bn, _LANE), lambda i: (i, 0)),
        out_shape=jax.ShapeDtypeStruct((n_rows, _LANE), jnp.float32),
    )(tableT)


def _matmul_kernel(emb, W, b2d, n_tokens: int, bm: int):
    def mk(e_ref, w_ref, b_ref, o_ref):
        acc = jnp.dot(
            e_ref[:, :_EMBED], w_ref[...], preferred_element_type=jnp.float32
        )
        o_ref[...] = jnp.maximum(acc + b_ref[...], 0.0)

    return pl.pallas_call(
        mk,
        grid=(n_tokens // bm,),
        in_specs=[
            pl.BlockSpec((bm, _LANE), lambda i: (i, 0)),
            pl.BlockSpec((_EMBED, _OUT), lambda i: (0, 0)),
            pl.BlockSpec((1, _OUT), lambda i: (0, 0)),
        ],
        out_specs=pl.BlockSpec((bm, _OUT), lambda i: (i, 0)),
        out_shape=jax.ShapeDtypeStruct((n_tokens, _OUT), jnp.float32),
    )(emb, W, b2d)


def kernel(x, table, W, b):
    batch, hist = x.shape
    n_tokens = batch * hist
    n_rows = table.shape[0]
    idx = jnp.transpose(x).reshape(n_tokens)
    table128 = _transpose_pad_kernel(jnp.transpose(table), n_rows, 8192)
    emb = _gather_kernel(n_tokens, idx, table128)
    out = _matmul_kernel(emb, W, b.reshape(1, _OUT), n_tokens, 4096)
    return jnp.transpose(out.reshape(hist, batch, _OUT), (1, 0, 2))


# speedup vs baseline: 3.4453x; 1.0234x over previous
import functools

import jax
import jax.numpy as jnp
from jax import lax
from jax.experimental import pallas as pl
from jax.experimental.pallas import tpu as pltpu
from jax.experimental.pallas import tpu_sc as plsc

_NC = 2
_NS = 16
_NW = _NC * _NS

_EMBED = 64
_LANE = 128
_OUT = 512

_IDXW = 128
_SUB = 4
_CHUNK = _IDXW * _SUB
_N_CHUNKS = 5


def _gather_kernel(n_tokens: int, idx, table128):
    b_per_w = n_tokens // _NW
    n_chunks = b_per_w // _CHUNK
    mesh = plsc.VectorSubcoreMesh(core_axis_name="c", subcore_axis_name="s")

    @functools.partial(
        pl.kernel,
        out_type=jax.ShapeDtypeStruct((n_tokens, _LANE), jnp.float32),
        mesh=mesh,
        scratch_types=[
            pltpu.VMEM((_SUB, _IDXW), jnp.int32),
            pltpu.VMEM((_CHUNK, _LANE), jnp.float32),
            pltpu.SemaphoreType.DMA,
        ],
    )
    def gk(idx_hbm, table_hbm, out_hbm, idx_v, rows_v, sem):
        wid = lax.axis_index("s") * _NC + lax.axis_index("c")
        w_base = wid * b_per_w

        def body(i, carry):
            base = w_base + i * _CHUNK
            pltpu.sync_copy(
                idx_hbm.at[pl.ds(pl.multiple_of(base // _IDXW, _SUB), _SUB)],
                idx_v.at[...],
            )
            for j in range(_SUB):
                pltpu.async_copy(
                    table_hbm.at[idx_v.at[j]],
                    rows_v.at[pl.ds(j * _IDXW, _IDXW)],
                    sem,
                )
            for j in range(_SUB):
                pltpu.make_async_copy(
                    table_hbm.at[idx_v.at[j]],
                    rows_v.at[pl.ds(j * _IDXW, _IDXW)],
                    sem,
                ).wait()
            pltpu.sync_copy(rows_v.at[...], out_hbm.at[pl.ds(base, _CHUNK)])
            return carry

        lax.fori_loop(0, n_chunks, body, 0, unroll=False)

    return gk(idx.reshape(n_tokens // _IDXW, _IDXW), table128)


def _transpose_pad_kernel(tableT, n_rows: int, bn: int):
    def tk(t_ref, o_ref):
        t = jnp.transpose(t_ref[...], (1, 0))
        o_ref[...] = jnp.concatenate([t, t], axis=1)

    grid = (n_rows + bn - 1) // bn
    return pl.pallas_call(
        tk,
        grid=(grid,),
        in_specs=[pl.BlockSpec((_EMBED, bn), lambda i: (0, i))],
        out_specs=pl.BlockSpec((bn, _LANE), lambda i: (i, 0)),
        out_shape=jax.ShapeDtypeStruct((n_rows, _LANE), jnp.float32),
    )(tableT)


def _matmul_chunk(prev, emb_c, W, b2d, n_tokens: int, blk_base: int, bm: int):
    def mk(p_ref, e_ref, w_ref, b_ref, o_ref):
        del p_ref
        acc = jnp.dot(
            e_ref[:, :_EMBED], w_ref[...], preferred_element_type=jnp.float32
        )
        o_ref[...] = jnp.maximum(acc + b_ref[...], 0.0)

    chunk_rows = emb_c.shape[0]
    emb_specs = [
        pl.BlockSpec((bm, _LANE), lambda i: (i, 0)),
        pl.BlockSpec((_EMBED, _OUT), lambda i: (0, 0)),
        pl.BlockSpec((1, _OUT), lambda i: (0, 0)),
    ]
    if prev is None:
        def mk0(e_ref, w_ref, b_ref, o_ref):
            mk(None, e_ref, w_ref, b_ref, o_ref)

        body, in_specs, args, aliases = mk0, emb_specs, (emb_c, W, b2d), {}
    else:
        body = mk
        in_specs = [pl.BlockSpec(memory_space=pltpu.MemorySpace.HBM)] + emb_specs
        args = (prev, emb_c, W, b2d)
        aliases = {0: 0}
    return pl.pallas_call(
        body,
        grid=(chunk_rows // bm,),
        in_specs=in_specs,
        out_specs=pl.BlockSpec((bm, _OUT), lambda i: (i + blk_base, 0)),
        out_shape=jax.ShapeDtypeStruct((n_tokens, _OUT), jnp.float32),
        input_output_aliases=aliases,
    )(*args)


def kernel(x, table, W, b):
    batch, hist = x.shape
    n_tokens = batch * hist
    n_rows = table.shape[0]
    idx = jnp.transpose(x).reshape(n_tokens)
    table128 = _transpose_pad_kernel(jnp.transpose(table), n_rows, 8192)
    b2d = b.reshape(1, _OUT)
    bm = 4096
    chunk_tokens = n_tokens // _N_CHUNKS
    embs = [
        _gather_kernel(
            chunk_tokens,
            lax.slice(idx, (c * chunk_tokens,), ((c + 1) * chunk_tokens,)),
            table128,
        )
        for c in range(_N_CHUNKS)
    ]
    out = None
    for c in range(_N_CHUNKS):
        out = _matmul_chunk(
            out, embs[c], W, b2d, n_tokens, c * (chunk_tokens // bm), bm
        )
    return jnp.transpose(out.reshape(hist, batch, _OUT), (1, 0, 2))


# speedup vs baseline: 3.5306x; 1.0248x over previous
import functools

import jax
import jax.numpy as jnp
from jax import lax
from jax.experimental import pallas as pl
from jax.experimental.pallas import tpu as pltpu
from jax.experimental.pallas import tpu_sc as plsc

_NC = 2
_NS = 16
_NW = _NC * _NS

_EMBED = 64
_LANE = 128
_OUT = 512

_IDXW = 128
_SUB = 4
_CHUNK = _IDXW * _SUB
_CHUNK_TOKENS = (32768, 98304, 229376, 229376, 229376)


def _gather_kernel(n_tokens: int, idx, table128):
    b_per_w = n_tokens // _NW
    n_chunks = b_per_w // _CHUNK
    mesh = plsc.VectorSubcoreMesh(core_axis_name="c", subcore_axis_name="s")

    @functools.partial(
        pl.kernel,
        out_type=jax.ShapeDtypeStruct((n_tokens, _LANE), jnp.float32),
        mesh=mesh,
        scratch_types=[
            pltpu.VMEM((_SUB, _IDXW), jnp.int32),
            pltpu.VMEM((_CHUNK, _LANE), jnp.float32),
            pltpu.SemaphoreType.DMA,
        ],
    )
    def gk(idx_hbm, table_hbm, out_hbm, idx_v, rows_v, sem):
        wid = lax.axis_index("s") * _NC + lax.axis_index("c")
        w_base = wid * b_per_w

        def body(i, carry):
            base = w_base + i * _CHUNK
            pltpu.sync_copy(
                idx_hbm.at[pl.ds(pl.multiple_of(base // _IDXW, _SUB), _SUB)],
                idx_v.at[...],
            )
            for j in range(_SUB):
                pltpu.async_copy(
                    table_hbm.at[idx_v.at[j]],
                    rows_v.at[pl.ds(j * _IDXW, _IDXW)],
                    sem,
                )
            for j in range(_SUB):
                pltpu.make_async_copy(
                    table_hbm.at[idx_v.at[j]],
                    rows_v.at[pl.ds(j * _IDXW, _IDXW)],
                    sem,
                ).wait()
            pltpu.sync_copy(rows_v.at[...], out_hbm.at[pl.ds(base, _CHUNK)])
            return carry

        lax.fori_loop(0, n_chunks, body, 0, unroll=False)

    return gk(idx.reshape(n_tokens // _IDXW, _IDXW), table128)


def _transpose_pad_kernel(tableT, n_rows: int, bn: int):
    def tk(t_ref, o_ref):
        t = jnp.transpose(t_ref[...], (1, 0))
        o_ref[...] = jnp.concatenate([t, t], axis=1)

    grid = (n_rows + bn - 1) // bn
    return pl.pallas_call(
        tk,
        grid=(grid,),
        in_specs=[pl.BlockSpec((_EMBED, bn), lambda i: (0, i))],
        out_specs=pl.BlockSpec((bn, _LANE), lambda i: (i, 0)),
        out_shape=jax.ShapeDtypeStruct((n_rows, _LANE), jnp.float32),
    )(tableT)


def _matmul_chunk(prev, emb_c, W, b2d, n_tokens: int, blk_base: int, bm: int):
    def mk(p_ref, e_ref, w_ref, b_ref, o_ref):
        del p_ref
        acc = jnp.dot(
            e_ref[:, :_EMBED], w_ref[...], preferred_element_type=jnp.float32
        )
        o_ref[...] = jnp.maximum(acc + b_ref[...], 0.0)

    chunk_rows = emb_c.shape[0]
    emb_specs = [
        pl.BlockSpec((bm, _LANE), lambda i: (i, 0)),
        pl.BlockSpec((_EMBED, _OUT), lambda i: (0, 0)),
        pl.BlockSpec((1, _OUT), lambda i: (0, 0)),
    ]
    if prev is None:
        def mk0(e_ref, w_ref, b_ref, o_ref):
            mk(None, e_ref, w_ref, b_ref, o_ref)

        body, in_specs, args, aliases = mk0, emb_specs, (emb_c, W, b2d), {}
    else:
        body = mk
        in_specs = [pl.BlockSpec(memory_space=pltpu.MemorySpace.HBM)] + emb_specs
        args = (prev, emb_c, W, b2d)
        aliases = {0: 0}
    return pl.pallas_call(
        body,
        grid=(chunk_rows // bm,),
        in_specs=in_specs,
        out_specs=pl.BlockSpec((bm, _OUT), lambda i: (i + blk_base, 0)),
        out_shape=jax.ShapeDtypeStruct((n_tokens, _OUT), jnp.float32),
        input_output_aliases=aliases,
    )(*args)


def kernel(x, table, W, b):
    batch, hist = x.shape
    n_tokens = batch * hist
    n_rows = table.shape[0]
    idx = jnp.transpose(x).reshape(n_tokens)
    table128 = _transpose_pad_kernel(jnp.transpose(table), n_rows, 16384)
    b2d = b.reshape(1, _OUT)
    bm = 4096
    bases = [sum(_CHUNK_TOKENS[:c]) for c in range(len(_CHUNK_TOKENS))]
    embs = [
        _gather_kernel(
            ct, lax.slice(idx, (base,), (base + ct,)), table128
        )
        for base, ct in zip(bases, _CHUNK_TOKENS)
    ]
    out = None
    for base, emb_c in zip(bases, embs):
        out = _matmul_chunk(out, emb_c, W, b2d, n_tokens, base // bm, bm)
    return jnp.transpose(out.reshape(hist, batch, _OUT), (1, 0, 2))
